# baseline (speedup 1.0000x reference)
import functools

import jax
import jax.numpy as jnp
from jax import lax
from jax.experimental import pallas as pl
from jax.experimental.pallas import tpu as pltpu
from jax.experimental.pallas import tpu_sc as plsc

_EMB_DIM = 64
_B = 4096
_BASKET = 20
_NC = 2
_NS = 16
_NW = _NC * _NS
_BPW = _B // _NW
_U = 4
_ROWS = _U * _BASKET
_STEPS = _BPW // _U
_NBUF = 4
_LANES = 16
_DCOLS = _EMB_DIM // _LANES
_VPS = _ROWS // _LANES

_mesh = plsc.VectorSubcoreMesh(core_axis_name="c", subcore_axis_name="s")


@functools.partial(
    pl.kernel,
    mesh=_mesh,
    out_type=jax.ShapeDtypeStruct((_B, _EMB_DIM), jnp.float32),
    compiler_params=pltpu.CompilerParams(
        use_tc_tiling_on_sc=False, needs_layout_passes=False
    ),
    scratch_types=[
        pltpu.VMEM((_BPW, _BASKET), jnp.int32),
        pltpu.VMEM((_BPW * _BASKET,), jnp.int32),
        pltpu.VMEM((_NBUF, _ROWS, _EMB_DIM), jnp.float32),
        pltpu.VMEM((_BPW, _EMB_DIM), jnp.float32),
        [pltpu.SemaphoreType.DMA] * _NBUF,
    ],
)
def _basket_sum(s_hbm, table_hbm, out_hbm, stage_v, idx_v, rows_v, out_v, sems):
    wid = lax.axis_index("s") * _NC + lax.axis_index("c")
    ubase = wid * _BPW
    pltpu.sync_copy(
        s_hbm.at[pl.ds(ubase, _BPW), s_hbm.shape[1] - 1], stage_v
    )

    io = lax.iota(jnp.int32, _LANES)
    patterns = []
    for t in range(_VPS):
        p = io + t * _LANES
        rows = p // _BASKET
        cols = p - rows * _BASKET
        patterns.append((rows, cols))

    def repack(g, carry):
        for t in range(_VPS):
            rows, cols = patterns[t]
            v = plsc.load_gather(stage_v, [g * _U + rows, cols])
            idx_v[pl.ds(g * _ROWS + t * _LANES, _LANES)] = v
        return carry

    lax.fori_loop(0, _STEPS, repack, 0)

    def gather(s, b):
        return pltpu.make_async_copy(
            table_hbm.at[idx_v.at[pl.ds(s * _ROWS, _ROWS)]], rows_v.at[b], sems[b]
        )

    for b in range(_NBUF):
        gather(b, b).start()

    def outer(g, carry):
        for b in range(_NBUF):
            s = g * _NBUF + b
            gather(s, b).wait()
            for u in range(_U):
                for d in range(_DCOLS):
                    acc = rows_v[b, u * _BASKET, pl.ds(d * _LANES, _LANES)]
                    for j in range(1, _BASKET):
                        acc = acc + rows_v[
                            b, u * _BASKET + j, pl.ds(d * _LANES, _LANES)
                        ]
                    out_v[s * _U + u, pl.ds(d * _LANES, _LANES)] = acc

            @pl.when(s + _NBUF < _STEPS)
            def _():
                gather(s + _NBUF, b).start()

        return carry

    lax.fori_loop(0, _STEPS // _NBUF, outer, 0)
    pltpu.sync_copy(out_v, out_hbm.at[pl.ds(ubase, _BPW)])


def kernel(S, table):
    return _basket_sum(S, table)


# speedup vs baseline: 2.7948x; 2.7948x over previous
import functools

import jax
import jax.numpy as jnp
from jax import lax
from jax.experimental import pallas as pl
from jax.experimental.pallas import tpu as pltpu
from jax.experimental.pallas import tpu_sc as plsc

_EMB_DIM = 64
_B = 4096
_BASKET = 20
_NC = 2
_NS = 16
_NW = _NC * _NS
_BPW = _B // _NW
_U = 4
_ROWS = _U * _BASKET
_STEPS = _BPW // _U
_NBUF = 4
_LANES = 16
_DCOLS = _EMB_DIM // _LANES

_mesh = plsc.VectorSubcoreMesh(core_axis_name="c", subcore_axis_name="s")


@functools.partial(
    pl.kernel,
    mesh=_mesh,
    out_type=jax.ShapeDtypeStruct((_B, _EMB_DIM), jnp.float32),
    compiler_params=pltpu.CompilerParams(
        use_tc_tiling_on_sc=False, needs_layout_passes=False
    ),
    scratch_types=[
        pltpu.VMEM((_BPW * _BASKET,), jnp.int32),
        pltpu.VMEM((_NBUF, _ROWS, _EMB_DIM), jnp.float32),
        pltpu.VMEM((_BPW, _EMB_DIM), jnp.float32),
        [pltpu.SemaphoreType.DMA] * _NBUF,
    ],
)
def _basket_sum(idx_hbm, table_hbm, out_hbm, idx_v, rows_v, out_v, sems):
    wid = lax.axis_index("s") * _NC + lax.axis_index("c")
    ubase = wid * _BPW
    pltpu.sync_copy(idx_hbm.at[pl.ds(ubase * _BASKET, _BPW * _BASKET)], idx_v)

    def gather(s, b):
        return pltpu.make_async_copy(
            table_hbm.at[idx_v.at[pl.ds(s * _ROWS, _ROWS)]], rows_v.at[b], sems[b]
        )

    for b in range(_NBUF):
        gather(b, b).start()

    def outer(g, carry):
        for b in range(_NBUF):
            s = g * _NBUF + b
            gather(s, b).wait()
            for u in range(_U):
                for d in range(_DCOLS):
                    acc = rows_v[b, u * _BASKET, pl.ds(d * _LANES, _LANES)]
                    for j in range(1, _BASKET):
                        acc = acc + rows_v[
                            b, u * _BASKET + j, pl.ds(d * _LANES, _LANES)
                        ]
                    out_v[s * _U + u, pl.ds(d * _LANES, _LANES)] = acc

            @pl.when(s + _NBUF < _STEPS)
            def _():
                gather(s + _NBUF, b).start()

        return carry

    lax.fori_loop(0, _STEPS // _NBUF, outer, 0)
    pltpu.sync_copy(out_v, out_hbm.at[pl.ds(ubase, _BPW)])


def kernel(S, table):
    idx = S[:, -1, :].reshape(-1)
    return _basket_sum(idx, table)


# speedup vs baseline: 2.8019x; 1.0025x over previous
import functools

import jax
import jax.numpy as jnp
from jax import lax
from jax.experimental import pallas as pl
from jax.experimental.pallas import tpu as pltpu
from jax.experimental.pallas import tpu_sc as plsc

_EMB_DIM = 64
_B = 4096
_BASKET = 20
_NC = 2
_NS = 16
_NW = _NC * _NS
_BPW = _B // _NW
_U = 4
_ROWS = _U * _BASKET
_STEPS = _BPW // _U
_NBUF = 4
_LANES = 16
_DCOLS = _EMB_DIM // _LANES

_mesh = plsc.VectorSubcoreMesh(core_axis_name="c", subcore_axis_name="s")


@functools.partial(
    pl.kernel,
    mesh=_mesh,
    out_type=jax.ShapeDtypeStruct((_B * _EMB_DIM,), jnp.float32),
    compiler_params=pltpu.CompilerParams(
        use_tc_tiling_on_sc=False, needs_layout_passes=False
    ),
    scratch_types=[
        pltpu.VMEM((_BPW * _BASKET,), jnp.int32),
        pltpu.VMEM((_NBUF, _ROWS, _EMB_DIM), jnp.float32),
        pltpu.VMEM((_BPW * _EMB_DIM,), jnp.float32),
        [pltpu.SemaphoreType.DMA] * _NBUF,
    ],
)
def _basket_sum(idx_hbm, table_hbm, out_hbm, idx_v, rows_v, out_v, sems):
    wid = lax.axis_index("s") * _NC + lax.axis_index("c")
    ubase = wid * _BPW
    pltpu.sync_copy(idx_hbm.at[pl.ds(ubase * _BASKET, _BPW * _BASKET)], idx_v)

    def gather(s, b):
        return pltpu.make_async_copy(
            table_hbm.at[idx_v.at[pl.ds(s * _ROWS, _ROWS)]], rows_v.at[b], sems[b]
        )

    for b in range(_NBUF):
        gather(b, b).start()

    def outer(g, carry):
        for b in range(_NBUF):
            s = g * _NBUF + b
            gather(s, b).wait()
            for u in range(_U):
                for d in range(_DCOLS):
                    acc = rows_v[b, u * _BASKET, pl.ds(d * _LANES, _LANES)]
                    for j in range(1, _BASKET):
                        acc = acc + rows_v[
                            b, u * _BASKET + j, pl.ds(d * _LANES, _LANES)
                        ]
                    out_v[pl.ds((s * _U + u) * _EMB_DIM + d * _LANES, _LANES)] = acc

            @pl.when(s + _NBUF < _STEPS)
            def _():
                gather(s + _NBUF, b).start()

        return carry

    lax.fori_loop(0, _STEPS // _NBUF, outer, 0)
    pltpu.sync_copy(
        out_v, out_hbm.at[pl.ds(ubase * _EMB_DIM, _BPW * _EMB_DIM)]
    )


def kernel(S, table):
    idx = S[:, -1, :].reshape(-1)
    return _basket_sum(idx, table).reshape(_B, _EMB_DIM)


# speedup vs baseline: 2.8084x; 1.0023x over previous
import functools

import jax
import jax.numpy as jnp
from jax import lax
from jax.experimental import pallas as pl
from jax.experimental.pallas import tpu as pltpu
from jax.experimental.pallas import tpu_sc as plsc

_EMB_DIM = 64
_B = 4096
_BASKET = 20
_NC = 2
_NS = 16
_NW = _NC * _NS
_BPW = _B // _NW
_U = 4
_ROWS = _U * _BASKET
_STEPS = _BPW // _U
_NBUF = 4
_LANES = 16
_DCOLS = _EMB_DIM // _LANES

_mesh = plsc.VectorSubcoreMesh(core_axis_name="c", subcore_axis_name="s")


@functools.partial(
    pl.kernel,
    mesh=_mesh,
    out_type=jax.ShapeDtypeStruct((_B, _EMB_DIM), jnp.float32),
    compiler_params=pltpu.CompilerParams(
        use_tc_tiling_on_sc=False, needs_layout_passes=False
    ),
    scratch_types=[
        pltpu.VMEM((_BPW * _BASKET,), jnp.int32),
        pltpu.VMEM((_NBUF, _ROWS, _EMB_DIM), jnp.float32),
        pltpu.VMEM((_BPW, _EMB_DIM), jnp.float32),
        [pltpu.SemaphoreType.DMA] * _NBUF,
    ],
)
def _basket_sum(idx_hbm, table_hbm, out_hbm, idx_v, rows_v, out_v, sems):
    wid = lax.axis_index("s") * _NC + lax.axis_index("c")
    ubase = wid * _BPW
    pltpu.sync_copy(idx_hbm.at[pl.ds(ubase * _BASKET, _BPW * _BASKET)], idx_v)

    def gather(s, b):
        return pltpu.make_async_copy(
            table_hbm.at[idx_v.at[pl.ds(s * _ROWS, _ROWS)]], rows_v.at[b], sems[b]
        )

    for b in range(_NBUF):
        gather(b, b).start()

    def outer(g, carry):
        for b in range(_NBUF):
            s = g * _NBUF + b
            gather(s, b).wait()
            for u in range(_U):
                for d in range(_DCOLS):
                    acc = rows_v[b, u * _BASKET, pl.ds(d * _LANES, _LANES)]
                    for j in range(1, _BASKET):
                        acc = acc + rows_v[
                            b, u * _BASKET + j, pl.ds(d * _LANES, _LANES)
                        ]
                    out_v[s * _U + u, pl.ds(d * _LANES, _LANES)] = acc

            @pl.when(s + _NBUF < _STEPS)
            def _():
                gather(s + _NBUF, b).start()

        return carry

    lax.fori_loop(0, _STEPS // _NBUF, outer, 0)
    pltpu.sync_copy(out_v, out_hbm.at[pl.ds(ubase, _BPW)])


def kernel(S, table):
    idx = S[:, -1, :].reshape(-1)
    return _basket_sum(idx, table)
